# Initial kernel scaffold; baseline (speedup 1.0000x reference)
import jax
import jax.numpy as jnp
from jax.experimental import pallas as pl


def kernel(indices, table):
    raise NotImplementedError("write your pallas kernel here")


# speedup vs baseline: 1.1010x; 1.1010x over previous
import functools

import jax
import jax.numpy as jnp
from jax import lax
from jax.experimental import pallas as pl
from jax.experimental.pallas import tpu as pltpu
from jax.experimental.pallas import tpu_sc as plsc

_NUM_ROWS = 16384
_SEQ = 100
_DIM = 32
_TOTAL = _NUM_ROWS * _SEQ
_NC = 2
_NS = 16
_NW = _NC * _NS
_PER_W = _TOTAL // _NW
_CHUNK = 1024
_N_CHUNKS = _PER_W // _CHUNK


def _embed_body(idx_hbm, table_hbm, out_hbm, idx_v, rows_v, sem):
    wid = lax.axis_index("s") * _NC + lax.axis_index("c")
    base = wid * _PER_W

    def chunk(j, carry):
        off = base + j * _CHUNK
        pltpu.sync_copy(idx_hbm.at[pl.ds(off, _CHUNK)], idx_v)
        pltpu.async_copy(table_hbm.at[idx_v], rows_v, sem).wait()
        pltpu.sync_copy(rows_v, out_hbm.at[pl.ds(off, _CHUNK)])
        return carry

    lax.fori_loop(0, _N_CHUNKS, chunk, 0)


_embed = functools.partial(
    pl.kernel,
    out_type=jax.ShapeDtypeStruct((_TOTAL, _DIM), jnp.float32),
    mesh=plsc.VectorSubcoreMesh(core_axis_name="c", subcore_axis_name="s"),
    scratch_types=[
        pltpu.VMEM((_CHUNK,), jnp.int32),
        pltpu.VMEM((_CHUNK, _DIM), jnp.float32),
        pltpu.SemaphoreType.DMA,
    ],
    compiler_params=pltpu.CompilerParams(use_tc_tiling_on_sc=False),
)(_embed_body)


def kernel(indices, table):
    flat = indices.reshape(_TOTAL).astype(jnp.int32)
    out = _embed(flat, table)
    return out.reshape(_NUM_ROWS, _SEQ, _DIM)


# speedup vs baseline: 1.1118x; 1.0098x over previous
import functools

import jax
import jax.numpy as jnp
from jax import lax
from jax.experimental import pallas as pl
from jax.experimental.pallas import tpu as pltpu
from jax.experimental.pallas import tpu_sc as plsc

_NUM_ROWS = 16384
_SEQ = 100
_DIM = 32
_TOTAL = _NUM_ROWS * _SEQ
_NC = 2
_NS = 16
_NW = _NC * _NS
_PER_W = _TOTAL // _NW
_CHUNK = 512
_N_CHUNKS = _PER_W // _CHUNK
_NB = 4
_AHEAD = 2
_GROUPS = _N_CHUNKS // _NB


def _embed_body(idx_hbm, table_hbm, out_hbm, idx_v, rows, gsems, osems):
    wid = lax.axis_index("s") * _NC + lax.axis_index("c")
    base = wid * _PER_W

    pltpu.sync_copy(idx_hbm.at[pl.ds(base, _PER_W)], idx_v)

    def idx_slice(j):
        return idx_v.at[pl.ds(j * _CHUNK, _CHUNK)]

    def out_slice(j):
        return out_hbm.at[pl.ds(base + j * _CHUNK, _CHUNK)]

    def gather(j, b):
        return pltpu.make_async_copy(table_hbm.at[idx_slice(j)], rows[b],
                                     gsems.at[b])

    def store(j, b):
        return pltpu.make_async_copy(rows[b], out_slice(j), osems.at[b])

    for b in range(_AHEAD):
        gather(b, b).start()

    def group(g, carry):
        for b in range(_NB):
            j = g * _NB + b
            gather(j, b).wait()
            store(j, b).start()
            jf = j + _AHEAD
            bf = (b + _AHEAD) % _NB

            @pl.when(jf < _N_CHUNKS)
            def _():
                @pl.when(jf >= _NB)
                def _():
                    store(jf - _NB, bf).wait()

                gather(jf, bf).start()

        return carry

    lax.fori_loop(0, _GROUPS, group, 0)

    for b in range(_NB):
        store(_N_CHUNKS - _NB + b, b).wait()


_embed = functools.partial(
    pl.kernel,
    out_type=jax.ShapeDtypeStruct((_TOTAL, _DIM), jnp.float32),
    mesh=plsc.VectorSubcoreMesh(core_axis_name="c", subcore_axis_name="s"),
    scratch_types=[
        pltpu.VMEM((_PER_W,), jnp.int32),
        [pltpu.VMEM((_CHUNK, _DIM), jnp.float32) for _ in range(_NB)],
        pltpu.SemaphoreType.DMA((_NB,)),
        pltpu.SemaphoreType.DMA((_NB,)),
    ],
    compiler_params=pltpu.CompilerParams(use_tc_tiling_on_sc=False),
)(_embed_body)


def kernel(indices, table):
    flat = indices.reshape(_TOTAL).astype(jnp.int32)
    out = _embed(flat, table)
    return out.reshape(_NUM_ROWS, _SEQ, _DIM)


# speedup vs baseline: 1.8580x; 1.6711x over previous
import functools

import jax
import jax.numpy as jnp
from jax import lax
from jax.experimental import pallas as pl
from jax.experimental.pallas import tpu as pltpu
from jax.experimental.pallas import tpu_sc as plsc

_V = 1_000_000
_D = 32
_B = 16384
_S = 100
_NC = 2
_NS = 16
_NW = _NC * _NS
_L = 16

_G = _V // 4
_VT = 7813
_VT_FULL = _VT - 1
_JT = 13


def _lane_consts():
    lanes = []
    for k in range(8):
        c = lax.iota(jnp.int32, _L) + 16 * k
        lanes.append(c)
    return lanes


def _detile_body(tabt_hbm, tail2_hbm, tlin_hbm, stage_v, ot0_v, ot1_v):
    wid = lax.axis_index("s") * _NC + lax.axis_index("c")
    lanes = _lane_consts()
    rowv = [c & 31 for c in lanes]
    cdiv = [lax.shift_right_logical(c, 5) for c in lanes]

    def do_tile(tg):
        pltpu.sync_copy(tabt_hbm.at[:, pl.ds(128 * tg, 128)], stage_v)
        for tr4 in range(4):
            ot_v = ot0_v if tr4 % 2 == 0 else ot1_v

            def srow(s, carry):
                for k in range(8):
                    colv = cdiv[k] + (32 * tr4 + 4 * s)
                    ot_v[s, pl.ds(16 * k, 16)] = plsc.load_gather(
                        stage_v, [rowv[k], colv])
                return carry

            lax.fori_loop(0, 8, srow, 0)
            tr = 4 * tg + tr4
            pltpu.sync_copy(ot_v, tlin_hbm.at[pl.ds(8 * tr, 8), :])

    def unit(n, carry):
        do_tile(wid + 32 * n)
        return carry

    lax.fori_loop(0, _VT_FULL // 32, unit, 0)

    @pl.when(wid < 4)
    def _():
        do_tile(wid + (_VT_FULL // 32) * 32)

    @pl.when(wid == 4)
    def _():
        pltpu.sync_copy(tail2_hbm, stage_v.at[pl.ds(0, 16), :])
        pltpu.sync_copy(stage_v.at[pl.ds(0, 16), :],
                        tlin_hbm.at[pl.ds(_G - 16, 16), :])


_detile = functools.partial(
    pl.kernel,
    out_type=jax.ShapeDtypeStruct((_G, 128), jnp.float32),
    mesh=plsc.VectorSubcoreMesh(core_axis_name="c", subcore_axis_name="s"),
    scratch_types=[
        pltpu.VMEM((_D, 128), jnp.float32),
        pltpu.VMEM((8, 128), jnp.float32),
        pltpu.VMEM((8, 128), jnp.float32),
    ],
    compiler_params=pltpu.CompilerParams(use_tc_tiling_on_sc=True, needs_layout_passes=False),
)(_detile_body)


def _gather_body(idxt_hbm, tlin_hbm, out_hbm, idxt_v, glist_v, sel_v,
                 stage_v, otile_v, sem):
    wid = lax.axis_index("s") * _NC + lax.axis_index("c")
    lanes = _lane_consts()

    def do_unit(jt, ic, nj):
        pltpu.sync_copy(idxt_hbm.at[pl.ds(8 * jt, nj), pl.ds(128 * ic, 128)],
                        idxt_v.at[pl.ds(0, nj), :])
        for j_loc in range(nj):
            for k in range(8):
                v = idxt_v[j_loc, pl.ds(16 * k, 16)]
                glist_v[pl.ds(16 * k, 16)] = lax.shift_right_logical(v, 2)
                sel_v[pl.ds(16 * k, 16)] = lax.shift_left(v & 3, 5)
            pltpu.async_copy(tlin_hbm.at[glist_v], stage_v, sem).wait()

            def drow(d, carry):
                for k in range(8):
                    colv = sel_v[pl.ds(16 * k, 16)] + d
                    otile_v[d, pl.ds(16 * k, 16)] = plsc.load_gather(
                        stage_v, [lanes[k], colv])
                return carry

            lax.fori_loop(0, _D, drow, 0)
            j = 8 * jt + j_loc
            pltpu.sync_copy(otile_v, out_hbm.at[j, :, pl.ds(128 * ic, 128)])

    def unit(n, carry):
        u = 32 * n + wid
        do_unit(u // 128, u % 128, 8)
        return carry

    lax.fori_loop(0, 48, unit, 0)

    for m in range(4):
        do_unit(12, 4 * wid + m, 4)


_gather = functools.partial(
    pl.kernel,
    out_type=jax.ShapeDtypeStruct((_S, _D, _B), jnp.float32),
    mesh=plsc.VectorSubcoreMesh(core_axis_name="c", subcore_axis_name="s"),
    scratch_types=[
        pltpu.VMEM((8, 128), jnp.int32),
        pltpu.VMEM((128,), jnp.int32),
        pltpu.VMEM((128,), jnp.int32),
        pltpu.VMEM((128, 128), jnp.float32),
        pltpu.VMEM((_D, 128), jnp.float32),
        pltpu.SemaphoreType.DMA,
    ],
    compiler_params=pltpu.CompilerParams(use_tc_tiling_on_sc=True, needs_layout_passes=False),
)(_gather_body)


def kernel(indices, table):
    idxt = indices.astype(jnp.int32).T
    tabt = table.T
    tail2 = table[4 * (_G - 16):].reshape(16, 128)
    tlin = _detile(tabt, tail2)
    outt = _gather(idxt, tlin)
    return outt.transpose(2, 0, 1)


# speedup vs baseline: 2.3837x; 1.2830x over previous
import functools

import jax
import jax.numpy as jnp
from jax import lax
from jax.experimental import pallas as pl
from jax.experimental.pallas import tpu as pltpu
from jax.experimental.pallas import tpu_sc as plsc

_V = 1_000_000
_D = 32
_B = 16384
_S = 100
_NC = 2
_NS = 16
_L = 16

_G = _V // 4
_VT_FULL = 7812
_PAIRS = _VT_FULL // 64


def _lane_consts():
    return [lax.iota(jnp.int32, _L) + 16 * k for k in range(8)]


def _detile_body(tabt_hbm, tail2_hbm, tlin_hbm, st0, st1, ob0, ob1,
                 isems, osems):
    wid = lax.axis_index("s") * _NC + lax.axis_index("c")
    lanes = _lane_consts()
    rowv = [c & 31 for c in lanes]
    cdiv = [lax.shift_right_logical(c, 5) for c in lanes]

    def in_desc(tg, st, b):
        return pltpu.make_async_copy(tabt_hbm.at[:, pl.ds(128 * tg, 128)],
                                     st, isems.at[b])

    def out_desc(tg, ob, b):
        return pltpu.make_async_copy(ob, tlin_hbm.at[pl.ds(32 * tg, 32), :],
                                     osems.at[b])

    def assemble(st, ob):
        def rloop(r, carry):
            addend = lax.shift_left(lax.shift_right_logical(r, 3), 5) \
                + lax.shift_left(r & 7, 2)
            for k in range(8):
                ob[r, pl.ds(16 * k, 16)] = plsc.load_gather(
                    st, [rowv[k], cdiv[k] + addend])
            return carry

        lax.fori_loop(0, 32, rloop, 0)

    in_desc(wid, st0, 0).start()
    in_desc(wid + 32, st1, 1).start()

    def pair(m, carry):
        for half, (st, ob, b) in enumerate(((st0, ob0, 0), (st1, ob1, 1))):
            tg = wid + 64 * m + 32 * half
            in_desc(tg, st, b).wait()

            @pl.when(m > 0)
            def _():
                out_desc(tg - 64, ob, b).wait()

            assemble(st, ob)
            out_desc(tg, ob, b).start()

            @pl.when(m < _PAIRS - 1)
            def _():
                in_desc(tg + 64, st, b).start()

        return carry

    lax.fori_loop(0, _PAIRS, pair, 0)
    out_desc(wid + 64 * (_PAIRS - 1), ob0, 0).wait()
    out_desc(wid + 64 * (_PAIRS - 1) + 32, ob1, 1).wait()

    @pl.when(wid < 4)
    def _():
        tg = wid + _VT_FULL - 4
        pltpu.sync_copy(tabt_hbm.at[:, pl.ds(128 * tg, 128)], st0)
        assemble(st0, ob0)
        pltpu.sync_copy(ob0, tlin_hbm.at[pl.ds(32 * tg, 32), :])

    @pl.when(wid == 4)
    def _():
        pltpu.sync_copy(tail2_hbm, st0.at[pl.ds(0, 16), :])
        pltpu.sync_copy(st0.at[pl.ds(0, 16), :],
                        tlin_hbm.at[pl.ds(_G - 16, 16), :])


_detile = functools.partial(
    pl.kernel,
    out_type=jax.ShapeDtypeStruct((_G, 128), jnp.float32),
    mesh=plsc.VectorSubcoreMesh(core_axis_name="c", subcore_axis_name="s"),
    scratch_types=[
        pltpu.VMEM((_D, 128), jnp.float32),
        pltpu.VMEM((_D, 128), jnp.float32),
        pltpu.VMEM((_D, 128), jnp.float32),
        pltpu.VMEM((_D, 128), jnp.float32),
        pltpu.SemaphoreType.DMA((2,)),
        pltpu.SemaphoreType.DMA((2,)),
    ],
    compiler_params=pltpu.CompilerParams(use_tc_tiling_on_sc=True,
                                         needs_layout_passes=False),
)(_detile_body)


def _gather_body(idxt_hbm, tlin_hbm, out_hbm, idxt_v, gl0, gl1, se0, se1,
                 st0, st1, ot0, ot1, gsems, osems):
    wid = lax.axis_index("s") * _NC + lax.axis_index("c")
    lanes = _lane_consts()
    GL, SE, ST, OT = [gl0, gl1], [se0, se1], [st0, st1], [ot0, ot1]

    def g_desc(b):
        return pltpu.make_async_copy(tlin_hbm.at[GL[b]], ST[b], gsems.at[b])

    def o_desc(jt, ic, j, b):
        return pltpu.make_async_copy(
            OT[b], out_hbm.at[8 * jt + j, :, pl.ds(128 * ic, 128)],
            osems.at[b])

    def compute_g(j, b):
        for k in range(8):
            v = idxt_v[j, pl.ds(16 * k, 16)]
            GL[b][pl.ds(16 * k, 16)] = lax.shift_right_logical(v, 2)
            SE[b][pl.ds(16 * k, 16)] = lax.shift_left(v & 3, 5)

    def assemble(b):
        se, st, ot = SE[b], ST[b], OT[b]

        def dloop(d, carry):
            for k in range(8):
                ot[d, pl.ds(16 * k, 16)] = plsc.load_gather(
                    st, [lanes[k], se[pl.ds(16 * k, 16)] + d])
            return carry

        lax.fori_loop(0, _D, dloop, 0)

    def unit(jt, ic, nj, prev):
        pltpu.sync_copy(idxt_hbm.at[pl.ds(8 * jt, nj), pl.ds(128 * ic, 128)],
                        idxt_v.at[pl.ds(0, nj), :])
        compute_g(0, 0)
        g_desc(0).start()
        for j in range(nj):
            b = j % 2
            if j + 1 < nj:
                compute_g(j + 1, 1 - b)
                g_desc(1 - b).start()
            g_desc(b).wait()
            if j >= 2:
                o_desc(jt, ic, j - 2, b).wait()
            else:
                pjt, pic, pnj, guard = prev
                if guard is None:
                    o_desc(pjt, pic, pnj - 2 + j, b).wait()
                else:
                    @pl.when(guard)
                    def _():
                        o_desc(pjt, pic, pnj - 2 + j, b).wait()
            assemble(b)
            o_desc(jt, ic, j, b).start()

    def mainu(n, carry):
        u = 32 * n + wid
        up = u - 32
        unit(u // 128, u % 128, 8, (up // 128, up % 128, 8, n > 0))
        return carry

    lax.fori_loop(0, 48, mainu, 0)

    ulast = 32 * 47 + wid
    for m in range(4):
        if m == 0:
            prev = (ulast // 128, ulast % 128, 8, None)
        else:
            prev = (12, 4 * wid + (m - 1), 4, None)
        unit(12, 4 * wid + m, 4, prev)

    o_desc(12, 4 * wid + 3, 2, 0).wait()
    o_desc(12, 4 * wid + 3, 3, 1).wait()


_gather = functools.partial(
    pl.kernel,
    out_type=jax.ShapeDtypeStruct((_S, _D, _B), jnp.float32),
    mesh=plsc.VectorSubcoreMesh(core_axis_name="c", subcore_axis_name="s"),
    scratch_types=[
        pltpu.VMEM((8, 128), jnp.int32),
        pltpu.VMEM((128,), jnp.int32),
        pltpu.VMEM((128,), jnp.int32),
        pltpu.VMEM((128,), jnp.int32),
        pltpu.VMEM((128,), jnp.int32),
        pltpu.VMEM((128, 128), jnp.float32),
        pltpu.VMEM((128, 128), jnp.float32),
        pltpu.VMEM((_D, 128), jnp.float32),
        pltpu.VMEM((_D, 128), jnp.float32),
        pltpu.SemaphoreType.DMA((2,)),
        pltpu.SemaphoreType.DMA((2,)),
    ],
    compiler_params=pltpu.CompilerParams(use_tc_tiling_on_sc=True,
                                         needs_layout_passes=False),
)(_gather_body)


def kernel(indices, table):
    idxt = indices.astype(jnp.int32).T
    tabt = table.T
    tail2 = table[4 * (_G - 16):].reshape(16, 128)
    tlin = _detile(tabt, tail2)
    outt = _gather(idxt, tlin)
    return outt.transpose(2, 0, 1)


# speedup vs baseline: 3.1119x; 1.3055x over previous
import functools

import jax
import jax.numpy as jnp
from jax import lax
from jax.experimental import pallas as pl
from jax.experimental.pallas import tpu as pltpu
from jax.experimental.pallas import tpu_sc as plsc

_V = 1_000_000
_D = 32
_B = 16384
_S = 100
_NC = 2
_NS = 16
_L = 16

_G = _V // 4
_VT_FULL = 7812
_PAIRS = _VT_FULL // 64


def _lane_consts():
    return [lax.iota(jnp.int32, _L) + 16 * k for k in range(8)]


def _detile_body(tabt_hbm, tail2_hbm, tlin_hbm, st0, st1, ob0, ob1,
                 isems, osems):
    wid = lax.axis_index("s") * _NC + lax.axis_index("c")
    lanes = _lane_consts()
    rowv = [c & 31 for c in lanes]
    cdiv = [lax.shift_right_logical(c, 5) for c in lanes]

    def in_desc(tg, st, b):
        return pltpu.make_async_copy(tabt_hbm.at[:, pl.ds(128 * tg, 128)],
                                     st, isems.at[b])

    def out_desc(tg, ob, b):
        return pltpu.make_async_copy(ob, tlin_hbm.at[pl.ds(32 * tg, 32), :],
                                     osems.at[b])

    def assemble(st, ob):
        def rloop(r4, carry):
            for dr in range(4):
                r = 4 * r4 + dr
                addend = lax.shift_left(lax.shift_right_logical(r, 3), 5) \
                    + lax.shift_left(r & 7, 2)
                for k in range(8):
                    ob[r, pl.ds(16 * k, 16)] = plsc.load_gather(
                        st, [rowv[k], cdiv[k] + addend])
            return carry

        lax.fori_loop(0, 8, rloop, 0)

    in_desc(wid, st0, 0).start()
    in_desc(wid + 32, st1, 1).start()

    def pair(m, carry):
        for half, (st, ob, b) in enumerate(((st0, ob0, 0), (st1, ob1, 1))):
            tg = wid + 64 * m + 32 * half
            in_desc(tg, st, b).wait()

            @pl.when(m > 0)
            def _():
                out_desc(tg - 64, ob, b).wait()

            assemble(st, ob)
            out_desc(tg, ob, b).start()

            @pl.when(m < _PAIRS - 1)
            def _():
                in_desc(tg + 64, st, b).start()

        return carry

    lax.fori_loop(0, _PAIRS, pair, 0)
    out_desc(wid + 64 * (_PAIRS - 1), ob0, 0).wait()
    out_desc(wid + 64 * (_PAIRS - 1) + 32, ob1, 1).wait()

    @pl.when(wid < 4)
    def _():
        tg = wid + _VT_FULL - 4
        pltpu.sync_copy(tabt_hbm.at[:, pl.ds(128 * tg, 128)], st0)
        assemble(st0, ob0)
        pltpu.sync_copy(ob0, tlin_hbm.at[pl.ds(32 * tg, 32), :])

    @pl.when(wid == 4)
    def _():
        pltpu.sync_copy(tail2_hbm, st0.at[pl.ds(0, 16), :])
        pltpu.sync_copy(st0.at[pl.ds(0, 16), :],
                        tlin_hbm.at[pl.ds(_G - 16, 16), :])


_detile = functools.partial(
    pl.kernel,
    out_type=jax.ShapeDtypeStruct((_G, 128), jnp.float32),
    mesh=plsc.VectorSubcoreMesh(core_axis_name="c", subcore_axis_name="s"),
    scratch_types=[
        pltpu.VMEM((_D, 128), jnp.float32),
        pltpu.VMEM((_D, 128), jnp.float32),
        pltpu.VMEM((_D, 128), jnp.float32),
        pltpu.VMEM((_D, 128), jnp.float32),
        pltpu.SemaphoreType.DMA((2,)),
        pltpu.SemaphoreType.DMA((2,)),
    ],
    compiler_params=pltpu.CompilerParams(use_tc_tiling_on_sc=True,
                                         needs_layout_passes=False),
)(_detile_body)


def _gather_body(idxt_hbm, tlin_hbm, out_hbm, idxt_v, gl0, gl1, se0, se1,
                 st0, st1, ot0, ot1, gsems, osems):
    wid = lax.axis_index("s") * _NC + lax.axis_index("c")
    lanes = _lane_consts()
    GL, SE, ST, OT = [gl0, gl1], [se0, se1], [st0, st1], [ot0, ot1]

    def g_desc(b):
        return pltpu.make_async_copy(tlin_hbm.at[GL[b]], ST[b], gsems.at[b])

    def o_desc(jt, ic, j, b):
        return pltpu.make_async_copy(
            OT[b], out_hbm.at[8 * jt + j, :, pl.ds(128 * ic, 128)],
            osems.at[b])

    def compute_g(j, b):
        for k in range(8):
            v = idxt_v[j, pl.ds(16 * k, 16)]
            GL[b][pl.ds(16 * k, 16)] = lax.shift_right_logical(v, 2)
            SE[b][pl.ds(16 * k, 16)] = lax.shift_left(v & 3, 5)

    def assemble(b):
        se, st, ot = SE[b], ST[b], OT[b]

        def dloop(d4, carry):
            sl = [se[pl.ds(16 * k, 16)] for k in range(8)]
            for dd in range(4):
                d = 4 * d4 + dd
                for k in range(8):
                    ot[d, pl.ds(16 * k, 16)] = plsc.load_gather(
                        st, [lanes[k], sl[k] + d])
            return carry

        lax.fori_loop(0, _D // 4, dloop, 0)

    def unit(jt, ic, nj, prev):
        pltpu.sync_copy(idxt_hbm.at[pl.ds(8 * jt, nj), pl.ds(128 * ic, 128)],
                        idxt_v.at[pl.ds(0, nj), :])
        compute_g(0, 0)
        g_desc(0).start()
        for j in range(nj):
            b = j % 2
            if j + 1 < nj:
                compute_g(j + 1, 1 - b)
                g_desc(1 - b).start()
            g_desc(b).wait()
            if j >= 2:
                o_desc(jt, ic, j - 2, b).wait()
            else:
                pjt, pic, pnj, guard = prev
                if guard is None:
                    o_desc(pjt, pic, pnj - 2 + j, b).wait()
                else:
                    @pl.when(guard)
                    def _():
                        o_desc(pjt, pic, pnj - 2 + j, b).wait()
            assemble(b)
            o_desc(jt, ic, j, b).start()

    def mainu(n, carry):
        u = 32 * n + wid
        up = u - 32
        unit(u // 128, u % 128, 8, (up // 128, up % 128, 8, n > 0))
        return carry

    lax.fori_loop(0, 48, mainu, 0)

    ulast = 32 * 47 + wid
    for m in range(4):
        if m == 0:
            prev = (ulast // 128, ulast % 128, 8, None)
        else:
            prev = (12, 4 * wid + (m - 1), 4, None)
        unit(12, 4 * wid + m, 4, prev)

    o_desc(12, 4 * wid + 3, 2, 0).wait()
    o_desc(12, 4 * wid + 3, 3, 1).wait()


_gather = functools.partial(
    pl.kernel,
    out_type=jax.ShapeDtypeStruct((_S, _D, _B), jnp.float32),
    mesh=plsc.VectorSubcoreMesh(core_axis_name="c", subcore_axis_name="s"),
    scratch_types=[
        pltpu.VMEM((8, 128), jnp.int32),
        pltpu.VMEM((128,), jnp.int32),
        pltpu.VMEM((128,), jnp.int32),
        pltpu.VMEM((128,), jnp.int32),
        pltpu.VMEM((128,), jnp.int32),
        pltpu.VMEM((128, 128), jnp.float32),
        pltpu.VMEM((128, 128), jnp.float32),
        pltpu.VMEM((_D, 128), jnp.float32),
        pltpu.VMEM((_D, 128), jnp.float32),
        pltpu.SemaphoreType.DMA((2,)),
        pltpu.SemaphoreType.DMA((2,)),
    ],
    compiler_params=pltpu.CompilerParams(use_tc_tiling_on_sc=True,
                                         needs_layout_passes=False),
)(_gather_body)


def kernel(indices, table):
    idxt = indices.astype(jnp.int32).T
    tabt = table.T
    tail2 = table[4 * (_G - 16):].reshape(16, 128)
    tlin = _detile(tabt, tail2)
    outt = _gather(idxt, tlin)
    return outt.transpose(2, 0, 1)


# speedup vs baseline: 4.0407x; 1.2985x over previous
import functools

import jax
import jax.numpy as jnp
from jax import lax
from jax.experimental import pallas as pl
from jax.experimental.pallas import tpu as pltpu
from jax.experimental.pallas import tpu_sc as plsc

_V = 1_000_000
_D = 32
_B = 16384
_S = 100
_NC = 2
_NS = 16
_L = 16

_G = _V // 4
_VT_FULL = 7812
_PAIRS = _VT_FULL // 64


def _lane_consts():
    return [lax.iota(jnp.int32, _L) + 16 * k for k in range(8)]


def _detile_body(tabt_hbm, tail2_hbm, tlin_hbm, st0, st1, ob0, ob1,
                 isems, osems):
    wid = lax.axis_index("s") * _NC + lax.axis_index("c")
    lanes = _lane_consts()
    rowv = [c & 31 for c in lanes]
    cdiv = [lax.shift_right_logical(c, 5) for c in lanes]

    def in_desc(tg, st, b):
        return pltpu.make_async_copy(tabt_hbm.at[:, pl.ds(128 * tg, 128)],
                                     st, isems.at[b])

    def out_desc(tg, ob, b):
        return pltpu.make_async_copy(ob, tlin_hbm.at[pl.ds(32 * tg, 32), :],
                                     osems.at[b])

    def assemble(st, ob):
        def rloop(r4, carry):
            for dr in range(4):
                r = 4 * r4 + dr
                addend = lax.shift_left(lax.shift_right_logical(r, 3), 5) \
                    + lax.shift_left(r & 7, 2)
                for k in range(8):
                    ob[r, pl.ds(16 * k, 16)] = plsc.load_gather(
                        st, [rowv[k], cdiv[k] + addend])
            return carry

        lax.fori_loop(0, 8, rloop, 0)

    in_desc(wid, st0, 0).start()
    in_desc(wid + 32, st1, 1).start()

    def pair(m, carry):
        for half, (st, ob, b) in enumerate(((st0, ob0, 0), (st1, ob1, 1))):
            tg = wid + 64 * m + 32 * half
            in_desc(tg, st, b).wait()

            @pl.when(m > 0)
            def _():
                out_desc(tg - 64, ob, b).wait()

            assemble(st, ob)
            out_desc(tg, ob, b).start()

            @pl.when(m < _PAIRS - 1)
            def _():
                in_desc(tg + 64, st, b).start()

        return carry

    lax.fori_loop(0, _PAIRS, pair, 0)
    out_desc(wid + 64 * (_PAIRS - 1), ob0, 0).wait()
    out_desc(wid + 64 * (_PAIRS - 1) + 32, ob1, 1).wait()

    @pl.when(wid < 4)
    def _():
        tg = wid + _VT_FULL - 4
        pltpu.sync_copy(tabt_hbm.at[:, pl.ds(128 * tg, 128)], st0)
        assemble(st0, ob0)
        pltpu.sync_copy(ob0, tlin_hbm.at[pl.ds(32 * tg, 32), :])

    @pl.when(wid == 4)
    def _():
        pltpu.sync_copy(tail2_hbm, st0.at[pl.ds(0, 16), :])
        pltpu.sync_copy(st0.at[pl.ds(0, 16), :],
                        tlin_hbm.at[pl.ds(_G - 16, 16), :])


_detile = functools.partial(
    pl.kernel,
    out_type=jax.ShapeDtypeStruct((_G, 128), jnp.float32),
    mesh=plsc.VectorSubcoreMesh(core_axis_name="c", subcore_axis_name="s"),
    scratch_types=[
        pltpu.VMEM((_D, 128), jnp.float32),
        pltpu.VMEM((_D, 128), jnp.float32),
        pltpu.VMEM((_D, 128), jnp.float32),
        pltpu.VMEM((_D, 128), jnp.float32),
        pltpu.SemaphoreType.DMA((2,)),
        pltpu.SemaphoreType.DMA((2,)),
    ],
    compiler_params=pltpu.CompilerParams(use_tc_tiling_on_sc=True,
                                         needs_layout_passes=False),
)(_detile_body)


def _gather_body(idxt_hbm, tlin_hbm, out_hbm, idxt_v, gl0, gl1, se0, se1,
                 st0, st1, ot0, ot1, gsems, osems):
    wid = lax.axis_index("s") * _NC + lax.axis_index("c")
    lanes = _lane_consts()
    GL, SE, ST, OT = [gl0, gl1], [se0, se1], [st0, st1], [ot0, ot1]

    def g_desc(b):
        return pltpu.make_async_copy(tlin_hbm.at[GL[b]], ST[b], gsems.at[b])

    def o_desc(jt, ic, j, b):
        return pltpu.make_async_copy(
            OT[b], out_hbm.at[8 * jt + j, :, pl.ds(128 * ic, 128)],
            osems.at[b])

    def compute_g(j, b):
        for k in range(8):
            v = idxt_v[j, pl.ds(16 * k, 16)]
            GL[b][pl.ds(16 * k, 16)] = lax.shift_right_logical(v, 2)
            SE[b][pl.ds(16 * k, 16)] = lax.shift_left(v & 3, 5)

    def assemble(b):
        se, st, ot = SE[b], ST[b], OT[b]

        @plsc.parallel_loop(0, _D, step=1)
        def _(d):
            for k in range(8):
                ot[d, pl.ds(16 * k, 16)] = plsc.load_gather(
                    st, [lanes[k], se[pl.ds(16 * k, 16)] + d])

    def unit(jt, ic, nj, prev):
        pltpu.sync_copy(idxt_hbm.at[pl.ds(8 * jt, nj), pl.ds(128 * ic, 128)],
                        idxt_v.at[pl.ds(0, nj), :])
        compute_g(0, 0)
        g_desc(0).start()
        for j in range(nj):
            b = j % 2
            if j + 1 < nj:
                compute_g(j + 1, 1 - b)
                g_desc(1 - b).start()
            g_desc(b).wait()
            if j >= 2:
                o_desc(jt, ic, j - 2, b).wait()
            else:
                pjt, pic, pnj, guard = prev
                if guard is None:
                    o_desc(pjt, pic, pnj - 2 + j, b).wait()
                else:
                    @pl.when(guard)
                    def _():
                        o_desc(pjt, pic, pnj - 2 + j, b).wait()
            assemble(b)
            o_desc(jt, ic, j, b).start()

    def mainu(n, carry):
        u = 32 * n + wid
        up = u - 32
        unit(u // 128, u % 128, 8, (up // 128, up % 128, 8, n > 0))
        return carry

    lax.fori_loop(0, 48, mainu, 0)

    ulast = 32 * 47 + wid
    for m in range(4):
        if m == 0:
            prev = (ulast // 128, ulast % 128, 8, None)
        else:
            prev = (12, 4 * wid + (m - 1), 4, None)
        unit(12, 4 * wid + m, 4, prev)

    o_desc(12, 4 * wid + 3, 2, 0).wait()
    o_desc(12, 4 * wid + 3, 3, 1).wait()


_gather = functools.partial(
    pl.kernel,
    out_type=jax.ShapeDtypeStruct((_S, _D, _B), jnp.float32),
    mesh=plsc.VectorSubcoreMesh(core_axis_name="c", subcore_axis_name="s"),
    scratch_types=[
        pltpu.VMEM((8, 128), jnp.int32),
        pltpu.VMEM((128,), jnp.int32),
        pltpu.VMEM((128,), jnp.int32),
        pltpu.VMEM((128,), jnp.int32),
        pltpu.VMEM((128,), jnp.int32),
        pltpu.VMEM((128, 128), jnp.float32),
        pltpu.VMEM((128, 128), jnp.float32),
        pltpu.VMEM((_D, 128), jnp.float32),
        pltpu.VMEM((_D, 128), jnp.float32),
        pltpu.SemaphoreType.DMA((2,)),
        pltpu.SemaphoreType.DMA((2,)),
    ],
    compiler_params=pltpu.CompilerParams(use_tc_tiling_on_sc=True,
                                         needs_layout_passes=False),
)(_gather_body)


def kernel(indices, table):
    idxt = indices.astype(jnp.int32).T
    tabt = table.T
    tail2 = table[4 * (_G - 16):].reshape(16, 128)
    tlin = _detile(tabt, tail2)
    outt = _gather(idxt, tlin)
    return outt.transpose(2, 0, 1)


# speedup vs baseline: 5.1213x; 1.2674x over previous
import functools

import jax
import jax.numpy as jnp
from jax import lax
from jax.experimental import pallas as pl
from jax.experimental.pallas import tpu as pltpu
from jax.experimental.pallas import tpu_sc as plsc

_V = 1_000_000
_D = 32
_B = 16384
_S = 100
_NC = 2
_NS = 16
_L = 16

_G = _V // 4
_VT_FULL = 7812
_PAIRS = _VT_FULL // 64


def _lane_consts():
    return [lax.iota(jnp.int32, _L) + 16 * k for k in range(8)]


def _detile_body(tabt_hbm, tail2_hbm, tlin_hbm, st0, st1, ob0, ob1,
                 isems, osems):
    wid = lax.axis_index("s") * _NC + lax.axis_index("c")
    lanes = _lane_consts()
    rowv = [c & 31 for c in lanes]
    cdiv = [lax.shift_right_logical(c, 5) for c in lanes]

    def in_desc(tg, st, b):
        return pltpu.make_async_copy(tabt_hbm.at[:, pl.ds(128 * tg, 128)],
                                     st, isems.at[b])

    def out_desc(tg, ob, b):
        return pltpu.make_async_copy(ob, tlin_hbm.at[pl.ds(32 * tg, 32), :],
                                     osems.at[b])

    def assemble(st, ob):
        @plsc.parallel_loop(0, 32, step=1)
        def _(r):
            addend = lax.shift_left(lax.shift_right_logical(r, 3), 5) \
                + lax.shift_left(r & 7, 2)
            for k in range(8):
                ob[r, pl.ds(16 * k, 16)] = plsc.load_gather(
                    st, [rowv[k], cdiv[k] + addend])

    in_desc(wid, st0, 0).start()
    in_desc(wid + 32, st1, 1).start()

    def pair(m, carry):
        for half, (st, ob, b) in enumerate(((st0, ob0, 0), (st1, ob1, 1))):
            tg = wid + 64 * m + 32 * half
            in_desc(tg, st, b).wait()

            @pl.when(m > 0)
            def _():
                out_desc(tg - 64, ob, b).wait()

            assemble(st, ob)
            out_desc(tg, ob, b).start()

            @pl.when(m < _PAIRS - 1)
            def _():
                in_desc(tg + 64, st, b).start()

        return carry

    lax.fori_loop(0, _PAIRS, pair, 0)
    out_desc(wid + 64 * (_PAIRS - 1), ob0, 0).wait()
    out_desc(wid + 64 * (_PAIRS - 1) + 32, ob1, 1).wait()

    @pl.when(wid < 4)
    def _():
        tg = wid + _VT_FULL - 4
        pltpu.sync_copy(tabt_hbm.at[:, pl.ds(128 * tg, 128)], st0)
        assemble(st0, ob0)
        pltpu.sync_copy(ob0, tlin_hbm.at[pl.ds(32 * tg, 32), :])

    @pl.when(wid == 4)
    def _():
        pltpu.sync_copy(tail2_hbm, st0.at[pl.ds(0, 16), :])
        pltpu.sync_copy(st0.at[pl.ds(0, 16), :],
                        tlin_hbm.at[pl.ds(_G - 16, 16), :])


_detile = functools.partial(
    pl.kernel,
    out_type=jax.ShapeDtypeStruct((_G, 128), jnp.float32),
    mesh=plsc.VectorSubcoreMesh(core_axis_name="c", subcore_axis_name="s"),
    scratch_types=[
        pltpu.VMEM((_D, 128), jnp.float32),
        pltpu.VMEM((_D, 128), jnp.float32),
        pltpu.VMEM((_D, 128), jnp.float32),
        pltpu.VMEM((_D, 128), jnp.float32),
        pltpu.SemaphoreType.DMA((2,)),
        pltpu.SemaphoreType.DMA((2,)),
    ],
    compiler_params=pltpu.CompilerParams(use_tc_tiling_on_sc=True,
                                         needs_layout_passes=False),
)(_detile_body)


def _gather_body(idxt_hbm, tlin_hbm, out_hbm, idxt_v, gl0, gl1, se0, se1,
                 st0, st1, ot0, ot1, gsems, osems):
    wid = lax.axis_index("s") * _NC + lax.axis_index("c")
    lanes = _lane_consts()
    GL, SE, ST, OT = [gl0, gl1], [se0, se1], [st0, st1], [ot0, ot1]

    def g_desc(b):
        return pltpu.make_async_copy(tlin_hbm.at[GL[b]], ST[b], gsems.at[b])

    def o_desc(jt, ic, j, b):
        return pltpu.make_async_copy(
            OT[b], out_hbm.at[8 * jt + j, :, pl.ds(128 * ic, 128)],
            osems.at[b])

    def compute_g(j, b):
        for k in range(8):
            v = idxt_v[j, pl.ds(16 * k, 16)]
            GL[b][pl.ds(16 * k, 16)] = lax.shift_right_logical(v, 2)
            SE[b][pl.ds(16 * k, 16)] = lax.shift_left(v & 3, 5)

    def assemble(b):
        se, st, ot = SE[b], ST[b], OT[b]

        @plsc.parallel_loop(0, _D, step=1)
        def _(d):
            for k in range(8):
                ot[d, pl.ds(16 * k, 16)] = plsc.load_gather(
                    st, [lanes[k], se[pl.ds(16 * k, 16)] + d])

    def unit(jt, ic, nj, prev):
        pltpu.sync_copy(idxt_hbm.at[pl.ds(8 * jt, nj), pl.ds(128 * ic, 128)],
                        idxt_v.at[pl.ds(0, nj), :])
        compute_g(0, 0)
        g_desc(0).start()
        for j in range(nj):
            b = j % 2
            if j + 1 < nj:
                compute_g(j + 1, 1 - b)
                g_desc(1 - b).start()
            g_desc(b).wait()
            if j >= 2:
                o_desc(jt, ic, j - 2, b).wait()
            else:
                pjt, pic, pnj, guard = prev
                if guard is None:
                    o_desc(pjt, pic, pnj - 2 + j, b).wait()
                else:
                    @pl.when(guard)
                    def _():
                        o_desc(pjt, pic, pnj - 2 + j, b).wait()
            assemble(b)
            o_desc(jt, ic, j, b).start()

    def mainu(n, carry):
        u = 32 * n + wid
        up = u - 32
        unit(u // 128, u % 128, 8, (up // 128, up % 128, 8, n > 0))
        return carry

    lax.fori_loop(0, 48, mainu, 0)

    ulast = 32 * 47 + wid
    for m in range(4):
        if m == 0:
            prev = (ulast // 128, ulast % 128, 8, None)
        else:
            prev = (12, 4 * wid + (m - 1), 4, None)
        unit(12, 4 * wid + m, 4, prev)

    o_desc(12, 4 * wid + 3, 2, 0).wait()
    o_desc(12, 4 * wid + 3, 3, 1).wait()


_gather = functools.partial(
    pl.kernel,
    out_type=jax.ShapeDtypeStruct((_S, _D, _B), jnp.float32),
    mesh=plsc.VectorSubcoreMesh(core_axis_name="c", subcore_axis_name="s"),
    scratch_types=[
        pltpu.VMEM((8, 128), jnp.int32),
        pltpu.VMEM((128,), jnp.int32),
        pltpu.VMEM((128,), jnp.int32),
        pltpu.VMEM((128,), jnp.int32),
        pltpu.VMEM((128,), jnp.int32),
        pltpu.VMEM((128, 128), jnp.float32),
        pltpu.VMEM((128, 128), jnp.float32),
        pltpu.VMEM((_D, 128), jnp.float32),
        pltpu.VMEM((_D, 128), jnp.float32),
        pltpu.SemaphoreType.DMA((2,)),
        pltpu.SemaphoreType.DMA((2,)),
    ],
    compiler_params=pltpu.CompilerParams(use_tc_tiling_on_sc=True,
                                         needs_layout_passes=False),
)(_gather_body)


def kernel(indices, table):
    idxt = indices.astype(jnp.int32).T
    tabt = table.T
    tail2 = table[4 * (_G - 16):].reshape(16, 128)
    tlin = _detile(tabt, tail2)
    outt = _gather(idxt, tlin)
    return outt.transpose(2, 0, 1)


# speedup vs baseline: 5.1466x; 1.0049x over previous
import functools

import jax
import jax.numpy as jnp
from jax import lax
from jax.experimental import pallas as pl
from jax.experimental.pallas import tpu as pltpu
from jax.experimental.pallas import tpu_sc as plsc

_V = 1_000_000
_D = 32
_B = 16384
_S = 100
_NC = 2
_NS = 16
_L = 16

_G = _V // 4
_VT_FULL = 7812
_PAIRS = _VT_FULL // 64


def _lane_consts():
    return [lax.iota(jnp.int32, _L) + 16 * k for k in range(8)]


def _detile_body(tabt_hbm, tail2_hbm, tlin_hbm, st0, st1, ob0, ob1,
                 isems, osems):
    wid = lax.axis_index("s") * _NC + lax.axis_index("c")
    lanes = _lane_consts()
    rowv = [c & 31 for c in lanes]
    cdiv = [lax.shift_right_logical(c, 5) for c in lanes]

    def in_desc(tg, st, b):
        return pltpu.make_async_copy(tabt_hbm.at[:, pl.ds(128 * tg, 128)],
                                     st, isems.at[b])

    def out_desc(tg, ob, b):
        return pltpu.make_async_copy(ob, tlin_hbm.at[pl.ds(32 * tg, 32), :],
                                     osems.at[b])

    def assemble(st, ob):
        @plsc.parallel_loop(0, 32, step=1, unroll=2)
        def _(r):
            addend = lax.shift_left(lax.shift_right_logical(r, 3), 5) \
                + lax.shift_left(r & 7, 2)
            for k in range(8):
                ob[r, pl.ds(16 * k, 16)] = plsc.load_gather(
                    st, [rowv[k], cdiv[k] + addend])

    in_desc(wid, st0, 0).start()
    in_desc(wid + 32, st1, 1).start()

    def pair(m, carry):
        for half, (st, ob, b) in enumerate(((st0, ob0, 0), (st1, ob1, 1))):
            tg = wid + 64 * m + 32 * half
            in_desc(tg, st, b).wait()

            @pl.when(m > 0)
            def _():
                out_desc(tg - 64, ob, b).wait()

            assemble(st, ob)
            out_desc(tg, ob, b).start()

            @pl.when(m < _PAIRS - 1)
            def _():
                in_desc(tg + 64, st, b).start()

        return carry

    lax.fori_loop(0, _PAIRS, pair, 0)
    out_desc(wid + 64 * (_PAIRS - 1), ob0, 0).wait()
    out_desc(wid + 64 * (_PAIRS - 1) + 32, ob1, 1).wait()

    @pl.when(wid < 4)
    def _():
        tg = wid + _VT_FULL - 4
        pltpu.sync_copy(tabt_hbm.at[:, pl.ds(128 * tg, 128)], st0)
        assemble(st0, ob0)
        pltpu.sync_copy(ob0, tlin_hbm.at[pl.ds(32 * tg, 32), :])

    @pl.when(wid == 4)
    def _():
        pltpu.sync_copy(tail2_hbm, st0.at[pl.ds(0, 16), :])
        pltpu.sync_copy(st0.at[pl.ds(0, 16), :],
                        tlin_hbm.at[pl.ds(_G - 16, 16), :])


_detile = functools.partial(
    pl.kernel,
    out_type=jax.ShapeDtypeStruct((_G, 128), jnp.float32),
    mesh=plsc.VectorSubcoreMesh(core_axis_name="c", subcore_axis_name="s"),
    scratch_types=[
        pltpu.VMEM((_D, 128), jnp.float32),
        pltpu.VMEM((_D, 128), jnp.float32),
        pltpu.VMEM((_D, 128), jnp.float32),
        pltpu.VMEM((_D, 128), jnp.float32),
        pltpu.SemaphoreType.DMA((2,)),
        pltpu.SemaphoreType.DMA((2,)),
    ],
    compiler_params=pltpu.CompilerParams(use_tc_tiling_on_sc=True,
                                         needs_layout_passes=False),
)(_detile_body)


def _gather_body(idxt_hbm, tlin_hbm, out_hbm, idxt_v, gl0, gl1, se0, se1,
                 st0, st1, ot0, ot1, gsems, osems):
    wid = lax.axis_index("s") * _NC + lax.axis_index("c")
    lanes = _lane_consts()
    GL, SE, ST, OT = [gl0, gl1], [se0, se1], [st0, st1], [ot0, ot1]

    def g_desc(b):
        return pltpu.make_async_copy(tlin_hbm.at[GL[b]], ST[b], gsems.at[b])

    def o_desc(jt, ic, j, b):
        return pltpu.make_async_copy(
            OT[b], out_hbm.at[8 * jt + j, :, pl.ds(128 * ic, 128)],
            osems.at[b])

    def compute_g(j, b):
        for k in range(8):
            v = idxt_v[j, pl.ds(16 * k, 16)]
            GL[b][pl.ds(16 * k, 16)] = lax.shift_right_logical(v, 2)
            SE[b][pl.ds(16 * k, 16)] = lax.shift_left(v & 3, 5)

    def assemble(b):
        se, st, ot = SE[b], ST[b], OT[b]

        @plsc.parallel_loop(0, _D, step=1, unroll=2)
        def _(d):
            for k in range(8):
                ot[d, pl.ds(16 * k, 16)] = plsc.load_gather(
                    st, [lanes[k], se[pl.ds(16 * k, 16)] + d])

    def unit(jt, ic, nj, prev):
        pltpu.sync_copy(idxt_hbm.at[pl.ds(8 * jt, nj), pl.ds(128 * ic, 128)],
                        idxt_v.at[pl.ds(0, nj), :])
        compute_g(0, 0)
        g_desc(0).start()
        for j in range(nj):
            b = j % 2
            if j + 1 < nj:
                compute_g(j + 1, 1 - b)
                g_desc(1 - b).start()
            g_desc(b).wait()
            if j >= 2:
                o_desc(jt, ic, j - 2, b).wait()
            else:
                pjt, pic, pnj, guard = prev
                if guard is None:
                    o_desc(pjt, pic, pnj - 2 + j, b).wait()
                else:
                    @pl.when(guard)
                    def _():
                        o_desc(pjt, pic, pnj - 2 + j, b).wait()
            assemble(b)
            o_desc(jt, ic, j, b).start()

    def mainu(n, carry):
        u = 32 * n + wid
        up = u - 32
        unit(u // 128, u % 128, 8, (up // 128, up % 128, 8, n > 0))
        return carry

    lax.fori_loop(0, 48, mainu, 0)

    ulast = 32 * 47 + wid
    for m in range(4):
        if m == 0:
            prev = (ulast // 128, ulast % 128, 8, None)
        else:
            prev = (12, 4 * wid + (m - 1), 4, None)
        unit(12, 4 * wid + m, 4, prev)

    o_desc(12, 4 * wid + 3, 2, 0).wait()
    o_desc(12, 4 * wid + 3, 3, 1).wait()


_gather = functools.partial(
    pl.kernel,
    out_type=jax.ShapeDtypeStruct((_S, _D, _B), jnp.float32),
    mesh=plsc.VectorSubcoreMesh(core_axis_name="c", subcore_axis_name="s"),
    scratch_types=[
        pltpu.VMEM((8, 128), jnp.int32),
        pltpu.VMEM((128,), jnp.int32),
        pltpu.VMEM((128,), jnp.int32),
        pltpu.VMEM((128,), jnp.int32),
        pltpu.VMEM((128,), jnp.int32),
        pltpu.VMEM((128, 128), jnp.float32),
        pltpu.VMEM((128, 128), jnp.float32),
        pltpu.VMEM((_D, 128), jnp.float32),
        pltpu.VMEM((_D, 128), jnp.float32),
        pltpu.SemaphoreType.DMA((2,)),
        pltpu.SemaphoreType.DMA((2,)),
    ],
    compiler_params=pltpu.CompilerParams(use_tc_tiling_on_sc=True,
                                         needs_layout_passes=False),
)(_gather_body)


def kernel(indices, table):
    idxt = indices.astype(jnp.int32).T
    tabt = table.T
    tail2 = table[4 * (_G - 16):].reshape(16, 128)
    tlin = _detile(tabt, tail2)
    outt = _gather(idxt, tlin)
    return outt.transpose(2, 0, 1)
